# baseline (speedup 1.0000x reference)
import functools

import jax
import jax.numpy as jnp
from jax import lax
from jax.experimental import pallas as pl
from jax.experimental.pallas import tpu as pltpu
from jax.experimental.pallas import tpu_sc as plsc

N_NODES = 10000
N_EDGES = 320000
LANES = 16
NT = 16
EPT = N_EDGES // NT
Q = 5
QS = EPT // Q
NPAD = N_NODES


def _sc_spmv_chains(x_pad, src3, dst3, w3):
    mesh = plsc.VectorSubcoreMesh(core_axis_name="c", subcore_axis_name="s")

    @functools.partial(
        pl.kernel,
        mesh=mesh,
        compiler_params=pltpu.CompilerParams(needs_layout_passes=False),
        out_type=jax.ShapeDtypeStruct((5 * NPAD,), jnp.float32),
        scratch_types=(
            [pltpu.VMEM((QS,), jnp.int32)] * Q
            + [pltpu.VMEM((QS,), jnp.int32)] * Q
            + [pltpu.VMEM((QS,), jnp.float32)] * Q
            + [pltpu.VMEM((QS,), jnp.float32)] * Q
            + [
                pltpu.VMEM((NPAD,), jnp.float32),
                pltpu.VMEM_SHARED((NPAD,), jnp.float32),
                pltpu.VMEM_SHARED((NPAD,), jnp.float32),
                pltpu.VMEM_SHARED((NPAD,), jnp.float32),
                pltpu.SemaphoreType.DMA,
                pltpu.SemaphoreType.DMA,
            ]
        ),
    )
    def spmv_kernel(x_hbm, src_hbm, dst_hbm, w_hbm, out_hbm, *scratch):
        src_v = scratch[0:Q]
        dst_v = scratch[Q:2 * Q]
        w_v = scratch[2 * Q:3 * Q]
        msg_v = scratch[3 * Q:4 * Q]
        v_v, acc_a, acc_b, acc_c, sem_st, sem_sc = scratch[4 * Q:]
        cid = lax.axis_index("c")
        sid = lax.axis_index("s")

        base = sid * EPT
        stage = [pltpu.async_copy(x_hbm, v_v, sem_st)]
        for q in range(Q):
            qsl = pl.ds(base + q * QS, QS)
            stage.append(pltpu.async_copy(src_hbm.at[qsl], src_v[q], sem_st))
            stage.append(pltpu.async_copy(dst_hbm.at[qsl], dst_v[q], sem_st))
            stage.append(pltpu.async_copy(w_hbm.at[qsl], w_v[q], sem_st))

        @pl.when(sid < 3)
        def _():
            def zero_body(i, _):
                msg_v[0][pl.ds(i * LANES, LANES)] = jnp.zeros(
                    (LANES,), jnp.float32)
                return 0
            lax.fori_loop(0, QS // LANES, zero_body, 0, unroll=4)

        def zero_acc(acc):
            pltpu.sync_copy(msg_v[0], acc.at[pl.ds(0, QS)])
            pltpu.sync_copy(msg_v[0], acc.at[pl.ds(QS, QS)])
            pltpu.sync_copy(msg_v[0].at[pl.ds(0, NPAD - 2 * QS)],
                            acc.at[pl.ds(2 * QS, NPAD - 2 * QS)])

        @pl.when(sid == 0)
        def _():
            zero_acc(acc_a)

        @pl.when(sid == 1)
        def _():
            zero_acc(acc_b)

        @pl.when(sid == 2)
        def _():
            zero_acc(acc_c)

        for d in stage:
            d.wait()

        def spmv_pass(acc):
            descs = []
            for q in range(Q):
                sq, wq, mq = src_v[q], w_v[q], msg_v[q]

                @plsc.parallel_loop(0, QS, step=LANES, unroll=4)
                def body(i):
                    sl = pl.ds(i, LANES)
                    vals = plsc.load_gather(v_v, [sq[sl]])
                    mq[sl] = vals * wq[sl]
                descs.append(pltpu.async_copy(
                    mq, acc.at[dst_v[q]], sem_sc, add=True))
            for d in descs:
                d.wait()

        plsc.subcore_barrier()

        @pl.when(cid == 0)
        def _():
            spmv_pass(acc_a)

        @pl.when(cid == 1)
        def _():
            descs = [pltpu.async_copy(w_v[q], acc_a.at[dst_v[q]],
                                      sem_sc, add=True) for q in range(Q)]
            for d in descs:
                d.wait()

        plsc.subcore_barrier()

        PUB = NPAD // 5
        pub1 = jnp.where(cid == 0, 3 * NPAD, NPAD)

        def publish(acc, hbm_off):
            @pl.when(sid < 5)
            def _():
                sl = pl.ds(sid * PUB, PUB)
                pltpu.sync_copy(acc.at[sl], v_v.at[sl])
                pltpu.sync_copy(v_v.at[sl],
                                out_hbm.at[pl.ds(hbm_off + sid * PUB, PUB)])

        publish(acc_a, pub1)
        plsc.subcore_barrier()

        pltpu.sync_copy(out_hbm.at[pl.ds(pub1, NPAD)], v_v)
        spmv_pass(acc_b)

        plsc.subcore_barrier()

        @pl.when(cid == 0)
        def _():
            publish(acc_b, 4 * NPAD)
        plsc.subcore_barrier()

        @pl.when(cid == 0)
        def _():
            pltpu.sync_copy(out_hbm.at[pl.ds(4 * NPAD, NPAD)], v_v)
            spmv_pass(acc_c)

        plsc.subcore_barrier()

        @pl.when(cid == 0)
        def _():
            publish(acc_c, 0)

        @pl.when(cid == 1)
        def _():
            publish(acc_b, 2 * NPAD)

    return spmv_kernel(x_pad, src3, dst3, w3)


def _tc_expand(sums2d, W1, b1, W2, b2, W3, b3):
    def expand_kernel(sums_ref, w1_ref, b1_ref, w2_ref, b2_ref, w3_ref, b3_ref,
                      out_ref):
        sums2d = sums_ref[0:3, :]
        c1 = jnp.dot(jnp.dot(w1_ref[...], w2_ref[...],
                             preferred_element_type=jnp.float32), w3_ref[...],
                     preferred_element_type=jnp.float32)
        c2 = jnp.dot(jnp.dot(b1_ref[...], w2_ref[...],
                             preferred_element_type=jnp.float32), w3_ref[...],
                     preferred_element_type=jnp.float32)
        c3 = jnp.dot(b2_ref[...], w3_ref[...],
                     preferred_element_type=jnp.float32)
        coeff = jnp.concatenate([c1, c3, c2], axis=0)
        h = jax.lax.dot_general(
            sums2d, coeff, (((0,), (0,)), ((), ())),
            preferred_element_type=jnp.float32)
        h = h + b3_ref[...]
        out_ref[...] = 1.0 / (1.0 + jnp.exp(-h))

    return pl.pallas_call(
        expand_kernel,
        out_shape=jax.ShapeDtypeStruct((N_NODES, 128), jnp.float32),
    )(sums2d, W1, b1, W2, b2, W3, b3)


def kernel(x, edge_index, edge_weights, W1, b1, W2, b2, W3, b3):
    src = edge_index[0].astype(jnp.int32)
    dst = edge_index[1].astype(jnp.int32)
    w = edge_weights.astype(jnp.float32)
    x_flat = x[:, 0]

    sums = _sc_spmv_chains(x_flat, src, dst, w)
    sums2d = sums.reshape(5, NPAD)

    return _tc_expand(sums2d, W1, b1.reshape(1, -1), W2, b2.reshape(1, -1),
                      W3, b3.reshape(1, -1))


# speedup vs baseline: 1.0478x; 1.0478x over previous
import functools

import jax
import jax.numpy as jnp
from jax import lax
from jax.experimental import pallas as pl
from jax.experimental.pallas import tpu as pltpu
from jax.experimental.pallas import tpu_sc as plsc

N_NODES = 10000
N_EDGES = 320000
LANES = 16
NT = 16
EPT = N_EDGES // NT
Q = 5
QS = EPT // Q
NPAD = N_NODES


def _sc_spmv_chains(x_pad, src3, dst3, w3):
    mesh = plsc.VectorSubcoreMesh(core_axis_name="c", subcore_axis_name="s")

    @functools.partial(
        pl.kernel,
        mesh=mesh,
        compiler_params=pltpu.CompilerParams(needs_layout_passes=False),
        out_type=jax.ShapeDtypeStruct((3 * NPAD,), jnp.float32),
        scratch_types=(
            [pltpu.VMEM((QS,), jnp.int32)] * Q
            + [pltpu.VMEM((QS,), jnp.int32)] * Q
            + [pltpu.VMEM((QS,), jnp.float32)] * Q
            + [pltpu.VMEM((QS,), jnp.float32)] * Q
            + [
                pltpu.VMEM((NPAD,), jnp.float32),
                pltpu.VMEM_SHARED((NPAD,), jnp.float32),
                pltpu.VMEM_SHARED((NPAD,), jnp.float32),
                pltpu.VMEM_SHARED((NPAD,), jnp.float32),
                pltpu.SemaphoreType.DMA,
                pltpu.SemaphoreType.DMA,
            ]
        ),
    )
    def spmv_kernel(x_hbm, src_hbm, dst_hbm, w_hbm, out_hbm, *scratch):
        src_v = scratch[0:Q]
        dst_v = scratch[Q:2 * Q]
        w_v = scratch[2 * Q:3 * Q]
        msg_v = scratch[3 * Q:4 * Q]
        v_v, acc_a, acc_b, acc_c, sem_st, sem_sc = scratch[4 * Q:]
        cid = lax.axis_index("c")
        sid = lax.axis_index("s")

        base = sid * EPT
        stage = [pltpu.async_copy(x_hbm, v_v, sem_st)]
        for q in range(Q):
            qsl = pl.ds(base + q * QS, QS)
            stage.append(pltpu.async_copy(src_hbm.at[qsl], src_v[q], sem_st))
            stage.append(pltpu.async_copy(dst_hbm.at[qsl], dst_v[q], sem_st))
            stage.append(pltpu.async_copy(w_hbm.at[qsl], w_v[q], sem_st))

        @pl.when(sid < 3)
        def _():
            def zero_body(i, _):
                msg_v[0][pl.ds(i * LANES, LANES)] = jnp.zeros(
                    (LANES,), jnp.float32)
                return 0
            lax.fori_loop(0, QS // LANES, zero_body, 0, unroll=4)

        def zero_acc(acc):
            pltpu.sync_copy(msg_v[0], acc.at[pl.ds(0, QS)])
            pltpu.sync_copy(msg_v[0], acc.at[pl.ds(QS, QS)])
            pltpu.sync_copy(msg_v[0].at[pl.ds(0, NPAD - 2 * QS)],
                            acc.at[pl.ds(2 * QS, NPAD - 2 * QS)])

        @pl.when(sid == 0)
        def _():
            zero_acc(acc_a)

        @pl.when(sid == 1)
        def _():
            zero_acc(acc_b)

        @pl.when(sid == 2)
        def _():
            zero_acc(acc_c)

        for d in stage:
            d.wait()

        def spmv_pass(acc):
            descs = []
            for q in range(Q):
                sq, wq, mq = src_v[q], w_v[q], msg_v[q]

                @plsc.parallel_loop(0, QS, step=LANES, unroll=4)
                def body(i):
                    sl = pl.ds(i, LANES)
                    vals = plsc.load_gather(v_v, [sq[sl]])
                    mq[sl] = vals * wq[sl]
                descs.append(pltpu.async_copy(
                    mq, acc.at[dst_v[q]], sem_sc, add=True))
            for d in descs:
                d.wait()

        plsc.subcore_barrier()

        @pl.when(cid == 0)
        def _():
            spmv_pass(acc_a)

        @pl.when(cid == 1)
        def _():
            descs = [pltpu.async_copy(w_v[q], acc_a.at[dst_v[q]],
                                      sem_sc, add=True) for q in range(Q)]
            for d in descs:
                d.wait()

        plsc.subcore_barrier()

        pltpu.sync_copy(acc_a, v_v)
        spmv_pass(acc_b)

        plsc.subcore_barrier()

        @pl.when(cid == 0)
        def _():
            pltpu.sync_copy(acc_b, v_v)
            spmv_pass(acc_c)

        plsc.subcore_barrier()

        @pl.when(jnp.logical_and(cid == 0, sid == 0))
        def _():
            pltpu.sync_copy(acc_c, v_v)
            pltpu.sync_copy(v_v, out_hbm.at[pl.ds(0, NPAD)])

        @pl.when(jnp.logical_and(cid == 1, sid == 0))
        def _():
            pltpu.sync_copy(acc_a, v_v)
            pltpu.sync_copy(v_v, out_hbm.at[pl.ds(NPAD, NPAD)])
            pltpu.sync_copy(acc_b, v_v)
            pltpu.sync_copy(v_v, out_hbm.at[pl.ds(2 * NPAD, NPAD)])

    return spmv_kernel(x_pad, src3, dst3, w3)


def _tc_expand(sums2d, W1, b1, W2, b2, W3, b3):
    def expand_kernel(sums_ref, w1_ref, b1_ref, w2_ref, b2_ref, w3_ref, b3_ref,
                      out_ref):
        sums2d = sums_ref[...]
        c1 = jnp.dot(jnp.dot(w1_ref[...], w2_ref[...],
                             preferred_element_type=jnp.float32), w3_ref[...],
                     preferred_element_type=jnp.float32)
        c2 = jnp.dot(jnp.dot(b1_ref[...], w2_ref[...],
                             preferred_element_type=jnp.float32), w3_ref[...],
                     preferred_element_type=jnp.float32)
        c3 = jnp.dot(b2_ref[...], w3_ref[...],
                     preferred_element_type=jnp.float32)
        coeff = jnp.concatenate([c1, c3, c2], axis=0)
        h = jax.lax.dot_general(
            sums2d, coeff, (((0,), (0,)), ((), ())),
            preferred_element_type=jnp.float32)
        h = h + b3_ref[...]
        out_ref[...] = 1.0 / (1.0 + jnp.exp(-h))

    return pl.pallas_call(
        expand_kernel,
        out_shape=jax.ShapeDtypeStruct((N_NODES, 128), jnp.float32),
    )(sums2d, W1, b1, W2, b2, W3, b3)


def kernel(x, edge_index, edge_weights, W1, b1, W2, b2, W3, b3):
    src = edge_index[0].astype(jnp.int32)
    dst = edge_index[1].astype(jnp.int32)
    w = edge_weights.astype(jnp.float32)
    x_flat = x[:, 0]

    sums = _sc_spmv_chains(x_flat, src, dst, w)
    sums2d = sums.reshape(3, NPAD)

    return _tc_expand(sums2d, W1, b1.reshape(1, -1), W2, b2.reshape(1, -1),
                      W3, b3.reshape(1, -1))
